# speedup vs baseline: 2.9633x; 2.9633x over previous
import functools

import jax
import jax.numpy as jnp
from jax import lax
from jax.experimental import pallas as pl
from jax.experimental.pallas import tpu as pltpu
from jax.experimental.pallas import tpu_sc as plsc

_VOCAB = 100000
_D = 128
_B = 4096
_H = 50
_N = _B * _H
_ROWS_PER_DMA = 128


@functools.lru_cache(maxsize=None)
def _build(nc, ns):
    nw = nc * ns
    per_w = _N // nw
    nchunk = per_w // _ROWS_PER_DMA
    mesh = plsc.VectorSubcoreMesh(core_axis_name="c", subcore_axis_name="s")

    @functools.partial(
        pl.kernel,
        mesh=mesh,
        out_type=jax.ShapeDtypeStruct((_N, _D), jnp.float32),
        scratch_types=[
            pltpu.VMEM((nchunk, _ROWS_PER_DMA), jnp.int32),
            pltpu.VMEM((_ROWS_PER_DMA, _D), jnp.float32),
            pltpu.SemaphoreType.DMA,
        ],
    )
    def k(x_hbm, table_hbm, out_hbm, idx_v, buf, sem):
        wid = lax.axis_index("s") * nc + lax.axis_index("c")
        pltpu.sync_copy(x_hbm.at[wid], idx_v)

        def chunk(c, carry):
            pltpu.async_copy(table_hbm.at[idx_v.at[c]], buf, sem).wait()
            pltpu.sync_copy(
                buf,
                out_hbm.at[pl.ds(wid * per_w + c * _ROWS_PER_DMA, _ROWS_PER_DMA)],
            )
            return carry

        lax.fori_loop(0, nchunk, chunk, 0)

    return k


def kernel(x, table):
    info = plsc.get_sparse_core_info()
    nc, ns = info.num_cores, info.num_subcores
    nw = nc * ns
    xi = x.astype(jnp.int32).reshape(nw, _N // nw // _ROWS_PER_DMA, _ROWS_PER_DMA)
    out = _build(nc, ns)(xi, table)
    return out.reshape(_B, _H, _D)


# speedup vs baseline: 3.3353x; 1.1255x over previous
import functools

import jax
import jax.numpy as jnp
from jax import lax
from jax.experimental import pallas as pl
from jax.experimental.pallas import tpu as pltpu
from jax.experimental.pallas import tpu_sc as plsc

_VOCAB = 100000
_D = 128
_B = 4096
_H = 50
_N = _B * _H
_ROWS = 128
_RING = 5
_LOOK = 2


@functools.lru_cache(maxsize=None)
def _build(nc, ns):
    nw = nc * ns
    per_w = _N // nw
    nchunk = per_w // _ROWS
    assert nchunk % _RING == 0
    mesh = plsc.VectorSubcoreMesh(core_axis_name="c", subcore_axis_name="s")

    @functools.partial(
        pl.kernel,
        mesh=mesh,
        out_type=jax.ShapeDtypeStruct((_N, _D), jnp.float32),
        scratch_types=(
            [pltpu.VMEM((nchunk, _ROWS), jnp.int32)]
            + [pltpu.VMEM((_ROWS, _D), jnp.float32) for _ in range(_RING)]
            + [pltpu.SemaphoreType.DMA for _ in range(2 * _RING)]
        ),
    )
    def k(x_hbm, table_hbm, out_hbm, idx_v, *scratch):
        bufs = scratch[:_RING]
        semg = scratch[_RING:2 * _RING]
        sems = scratch[2 * _RING:]
        wid = lax.axis_index("s") * nc + lax.axis_index("c")
        pltpu.sync_copy(x_hbm.at[wid], idx_v)
        out_base = wid * per_w

        def gather(c, slot):
            return pltpu.make_async_copy(
                table_hbm.at[idx_v.at[c]], bufs[slot], semg[slot])

        def scatter(c, slot):
            return pltpu.make_async_copy(
                bufs[slot],
                out_hbm.at[pl.ds(out_base + c * _ROWS, _ROWS)],
                sems[slot])

        for c in range(_LOOK):
            gather(c, c).start()

        def outer(o, carry):
            for b in range(_RING):
                c = o * _RING + b
                gb = (b + _LOOK) % _RING
                @pl.when(c >= _RING - _LOOK)
                def _():
                    scatter(c - (_RING - _LOOK), gb).wait()

                @pl.when(c + _LOOK < nchunk)
                def _():
                    gather(c + _LOOK, gb).start()

                gather(c, b).wait()
                scatter(c, b).start()
            return carry

        lax.fori_loop(0, nchunk // _RING, outer, 0)
        for c in range(nchunk - (_RING - _LOOK), nchunk):
            scatter(c, c % _RING).wait()

    return k


def kernel(x, table):
    info = plsc.get_sparse_core_info()
    nc, ns = info.num_cores, info.num_subcores
    nw = nc * ns
    xi = x.astype(jnp.int32).reshape(nw, _N // nw // _ROWS, _ROWS)
    out = _build(nc, ns)(xi, table)
    return out.reshape(_B, _H, _D)


# speedup vs baseline: 3.3395x; 1.0013x over previous
import functools

import jax
import jax.numpy as jnp
from jax import lax
from jax.experimental import pallas as pl
from jax.experimental.pallas import tpu as pltpu
from jax.experimental.pallas import tpu_sc as plsc

_VOCAB = 100000
_D = 128
_B = 4096
_H = 50
_N = _B * _H
_ROWS = 64
_RING = 10
_LOOK = 5


@functools.lru_cache(maxsize=None)
def _build(nc, ns):
    nw = nc * ns
    per_w = _N // nw
    nchunk = per_w // _ROWS
    assert nchunk % _RING == 0
    mesh = plsc.VectorSubcoreMesh(core_axis_name="c", subcore_axis_name="s")

    @functools.partial(
        pl.kernel,
        mesh=mesh,
        out_type=jax.ShapeDtypeStruct((_N, _D), jnp.float32),
        scratch_types=(
            [pltpu.VMEM((nchunk, _ROWS), jnp.int32)]
            + [pltpu.VMEM((_ROWS, _D), jnp.float32) for _ in range(_RING)]
            + [pltpu.SemaphoreType.DMA for _ in range(2 * _RING)]
        ),
    )
    def k(x_hbm, table_hbm, out_hbm, idx_v, *scratch):
        bufs = scratch[:_RING]
        semg = scratch[_RING:2 * _RING]
        sems = scratch[2 * _RING:]
        wid = lax.axis_index("s") * nc + lax.axis_index("c")
        pltpu.sync_copy(x_hbm.at[wid], idx_v)
        out_base = wid * per_w

        def gather(c, slot):
            return pltpu.make_async_copy(
                table_hbm.at[idx_v.at[c]], bufs[slot], semg[slot])

        def scatter(c, slot):
            return pltpu.make_async_copy(
                bufs[slot],
                out_hbm.at[pl.ds(out_base + c * _ROWS, _ROWS)],
                sems[slot])

        for c in range(_LOOK):
            gather(c, c).start()

        def outer(o, carry):
            for b in range(_RING):
                c = o * _RING + b
                gb = (b + _LOOK) % _RING
                @pl.when(c >= _RING - _LOOK)
                def _():
                    scatter(c - (_RING - _LOOK), gb).wait()

                @pl.when(c + _LOOK < nchunk)
                def _():
                    gather(c + _LOOK, gb).start()

                gather(c, b).wait()
                scatter(c, b).start()
            return carry

        lax.fori_loop(0, nchunk // _RING, outer, 0)
        for c in range(nchunk - (_RING - _LOOK), nchunk):
            scatter(c, c % _RING).wait()

    return k


def kernel(x, table):
    info = plsc.get_sparse_core_info()
    nc, ns = info.num_cores, info.num_subcores
    nw = nc * ns
    xi = x.astype(jnp.int32).reshape(nw, _N // nw // _ROWS, _ROWS)
    out = _build(nc, ns)(xi, table)
    return out.reshape(_B, _H, _D)


# speedup vs baseline: 5.9470x; 1.7808x over previous
import functools

import jax
import jax.numpy as jnp
from jax import lax
from jax.experimental import pallas as pl
from jax.experimental.pallas import tpu as pltpu
from jax.experimental.pallas import tpu_sc as plsc

_VOCAB = 100000
_D = 128
_B = 4096
_H = 50
_N = _B * _H
_PLANES = 4
_CROWS = _PLANES * _H
_SPLITS = (0, 128)
_RING = 4
_LOOK = 2


@functools.lru_cache(maxsize=None)
def _build(nc, ns):
    nw = nc * ns
    per_w = _N // nw
    bat_w = _B // nw
    nchunk = per_w // _CROWS
    assert nchunk % _RING == 0
    lens = tuple(
        (_SPLITS[i + 1] if i + 1 < len(_SPLITS) else _CROWS) - _SPLITS[i]
        for i in range(len(_SPLITS)))
    mesh = plsc.VectorSubcoreMesh(core_axis_name="c", subcore_axis_name="s")

    @functools.partial(
        pl.kernel,
        mesh=mesh,
        out_type=jax.ShapeDtypeStruct((_B, _H, _D), jnp.float32),
        scratch_types=(
            [pltpu.VMEM((per_w,), jnp.int32)]
            + [pltpu.VMEM((_CROWS, _D), jnp.float32) for _ in range(_RING)]
            + [pltpu.SemaphoreType.DMA for _ in range(2 * _RING)]
        ),
        compiler_params=pltpu.CompilerParams(use_tc_tiling_on_sc=True),
    )
    def k(x_hbm, table_hbm, out_hbm, idx_v, *scratch):
        bufs = scratch[:_RING]
        semg = scratch[_RING:2 * _RING]
        sems = scratch[2 * _RING:]
        wid = lax.axis_index("s") * nc + lax.axis_index("c")
        pltpu.sync_copy(x_hbm.at[wid], idx_v)
        b_base = wid * bat_w

        def gathers(c, slot):
            return [
                pltpu.make_async_copy(
                    table_hbm.at[idx_v.at[pl.ds(c * _CROWS + off, ln)]],
                    bufs[slot].at[pl.ds(off, ln)],
                    semg[slot])
                for off, ln in zip(_SPLITS, lens)
            ]

        def scatters(c, slot):
            return [
                pltpu.make_async_copy(
                    bufs[slot].at[pl.ds(j * _H, _H)],
                    out_hbm.at[b_base + c * _PLANES + j],
                    sems[slot])
                for j in range(_PLANES)
            ]

        for c in range(_LOOK):
            for g in gathers(c, c):
                g.start()

        def outer(o, carry):
            for b in range(_RING):
                c = o * _RING + b
                gb = (b + _LOOK) % _RING
                @pl.when(c >= _RING - _LOOK)
                def _():
                    for s in scatters(c - (_RING - _LOOK), gb):
                        s.wait()

                @pl.when(c + _LOOK < nchunk)
                def _():
                    for g in gathers(c + _LOOK, gb):
                        g.start()

                for g in gathers(c, b):
                    g.wait()
                for s in scatters(c, b):
                    s.start()
            return carry

        lax.fori_loop(0, nchunk // _RING, outer, 0)
        for c in range(nchunk - (_RING - _LOOK), nchunk):
            for s in scatters(c, c % _RING):
                s.wait()

    return k


def kernel(x, table):
    info = plsc.get_sparse_core_info()
    nc, ns = info.num_cores, info.num_subcores
    nw = nc * ns
    xi = x.astype(jnp.int32).reshape(nw, _N // nw)
    return _build(nc, ns)(xi, table)


# speedup vs baseline: 10.6794x; 1.7958x over previous
import functools

import jax
import jax.numpy as jnp
from jax import lax
from jax.experimental import pallas as pl
from jax.experimental.pallas import tpu as pltpu
from jax.experimental.pallas import tpu_sc as plsc

_VOCAB = 100000
_D = 128
_B = 4096
_H = 50
_N = _B * _H
_RING = 5
_LOOK = 2


@functools.lru_cache(maxsize=None)
def _build(nc, ns):
    nw = nc * ns
    bat_w = _B // nw
    per_w = _H * bat_w
    nchunk = _H
    assert nchunk % _RING == 0
    mesh = plsc.VectorSubcoreMesh(core_axis_name="c", subcore_axis_name="s")

    @functools.partial(
        pl.kernel,
        mesh=mesh,
        out_type=jax.ShapeDtypeStruct((_H, _B, _D), jnp.float32),
        scratch_types=(
            [pltpu.VMEM((per_w,), jnp.int32)]
            + [pltpu.VMEM((bat_w, _D), jnp.float32) for _ in range(_RING)]
            + [pltpu.SemaphoreType.DMA for _ in range(2 * _RING)]
        ),
    )
    def k(x_hbm, table_hbm, out_hbm, idx_v, *scratch):
        bufs = scratch[:_RING]
        semg = scratch[_RING:2 * _RING]
        sems = scratch[2 * _RING:]
        wid = lax.axis_index("s") * nc + lax.axis_index("c")
        pltpu.sync_copy(x_hbm.at[wid], idx_v)
        b_base = wid * bat_w

        def gather(c, slot):
            return pltpu.make_async_copy(
                table_hbm.at[idx_v.at[pl.ds(c * bat_w, bat_w)]],
                bufs[slot], semg[slot])

        def scatter(c, slot):
            return pltpu.make_async_copy(
                bufs[slot],
                out_hbm.at[c, pl.ds(b_base, bat_w)],
                sems[slot])

        for c in range(_LOOK):
            gather(c, c).start()

        def outer(o, carry):
            for b in range(_RING):
                c = o * _RING + b
                gb = (b + _LOOK) % _RING
                @pl.when(c >= _RING - _LOOK)
                def _():
                    scatter(c - (_RING - _LOOK), gb).wait()

                @pl.when(c + _LOOK < nchunk)
                def _():
                    gather(c + _LOOK, gb).start()

                gather(c, b).wait()
                scatter(c, b).start()
            return carry

        lax.fori_loop(0, nchunk // _RING, outer, 0)
        for c in range(nchunk - (_RING - _LOOK), nchunk):
            scatter(c, c % _RING).wait()

    return k


def kernel(x, table):
    info = plsc.get_sparse_core_info()
    nc, ns = info.num_cores, info.num_subcores
    nw = nc * ns
    bat_w = _B // nw
    xi = (x.astype(jnp.int32).T.reshape(_H, nw, bat_w)
          .transpose(1, 0, 2).reshape(nw, _H * bat_w))
    out = _build(nc, ns)(xi, table)
    return out.transpose(1, 0, 2)
